# Initial kernel scaffold; baseline (speedup 1.0000x reference)
import jax
import jax.numpy as jnp
from jax.experimental import pallas as pl


def kernel(data, neigh, depth):
    raise NotImplementedError("write your pallas kernel here")


# speedup vs baseline: 5.9619x; 5.9619x over previous
import jax
import jax.numpy as jnp
from jax import lax
from jax.experimental import pallas as pl
from jax.experimental.pallas import tpu as pltpu
from jax.experimental.pallas import tpu_sc as plsc
import functools

N_CORES = 2
N_SUBCORES = 16
NW = N_CORES * N_SUBCORES
LANES = 16
CH = 128


def _build_gather(B: int, C: int):
    n_full, tail = divmod(B, CH)
    n_base, rem = divmod(n_full, NW)
    tail_wid = n_full % NW

    mesh = plsc.VectorSubcoreMesh(
        core_axis_name="c", subcore_axis_name="s",
        num_cores=N_CORES, num_subcores=N_SUBCORES)

    scratch = [
        pltpu.VMEM((CH,), jnp.int32),
        pltpu.VMEM((CH, C), jnp.float32),
        pltpu.SemaphoreType.DMA,
    ]
    if tail:
        scratch += [
            pltpu.VMEM((tail,), jnp.int32),
            pltpu.VMEM((tail, C), jnp.float32),
        ]

    @functools.partial(
        pl.kernel,
        out_type=jax.ShapeDtypeStruct((B, C), jnp.float32),
        mesh=mesh,
        scratch_types=scratch,
        compiler_params=pltpu.CompilerParams(use_tc_tiling_on_sc=False),
    )
    def gather_kernel(table_hbm, idx_hbm, out_hbm, idx_v, rows_v, sem,
                      *tail_refs):
        wid = lax.axis_index("s") * N_CORES + lax.axis_index("c")
        n_my = n_base + (wid < rem).astype(jnp.int32)

        def chunk_body(j, carry):
            base = (wid + j * NW) * CH
            pltpu.sync_copy(idx_hbm.at[pl.ds(base, CH)], idx_v)
            for i in range(CH // LANES):
                sl = pl.ds(i * LANES, LANES)
                idx_v[sl] = idx_v[sl] + 1
            pltpu.async_copy(table_hbm.at[idx_v], rows_v, sem).wait()
            pltpu.sync_copy(rows_v, out_hbm.at[pl.ds(base, CH)])
            return carry

        lax.fori_loop(0, n_my, chunk_body, 0)

        if tail:
            idx_t, rows_t = tail_refs

            @pl.when(wid == tail_wid)
            def _():
                base = n_full * CH
                pltpu.sync_copy(idx_hbm.at[pl.ds(base, tail)], idx_t)
                for i in range(tail // LANES):
                    sl = pl.ds(i * LANES, LANES)
                    idx_t[sl] = idx_t[sl] + 1
                pltpu.async_copy(table_hbm.at[idx_t], rows_t, sem).wait()
                pltpu.sync_copy(rows_t, out_hbm.at[pl.ds(base, tail)])

    return gather_kernel


def kernel(data, neigh, depth):
    n, k = neigh.shape
    c = data.shape[1]
    table = jnp.concatenate([jnp.zeros((1, c), data.dtype), data], axis=0)
    idx = neigh.reshape(-1).astype(jnp.int32)
    out_flat = _build_gather(n * k, c)(table, idx)
    return out_flat.reshape(n, k, c)


# speedup vs baseline: 7.4362x; 1.2473x over previous
import jax
import jax.numpy as jnp
from jax import lax
from jax.experimental import pallas as pl
from jax.experimental.pallas import tpu as pltpu
from jax.experimental.pallas import tpu_sc as plsc
import functools

N_CORES = 2
N_SUBCORES = 16
NW = N_CORES * N_SUBCORES
LANES = 16
CH = 128
G = 8


def _build_gather(B: int, C: int):
    span = ((B + NW - 1) // NW + 7) // 8 * 8
    last = B - (NW - 1) * span
    assert 0 < last <= span
    n_main = (min(span, last) // CH) // G * G
    rem_hi = span - n_main * CH
    rem_lo = last - n_main * CH
    hi_full, hi_part = divmod(rem_hi, CH)
    lo_full, lo_part = divmod(rem_lo, CH)
    assert hi_full <= G and lo_full <= G
    assert hi_part % 8 == 0 and lo_part % 8 == 0

    mesh = plsc.VectorSubcoreMesh(
        core_axis_name="c", subcore_axis_name="s",
        num_cores=N_CORES, num_subcores=N_SUBCORES)

    scratch = [
        pltpu.VMEM((G, CH), jnp.int32),
        pltpu.VMEM((G, CH, C), jnp.float32),
        pltpu.SemaphoreType.DMA((G,)),
        pltpu.SemaphoreType.DMA((G,)),
    ]
    for part in (hi_part, lo_part):
        if part:
            scratch += [pltpu.VMEM((part,), jnp.int32),
                        pltpu.VMEM((part, C), jnp.float32)]

    @functools.partial(
        pl.kernel,
        out_type=jax.ShapeDtypeStruct((B, C), jnp.float32),
        mesh=mesh,
        scratch_types=scratch,
        compiler_params=pltpu.CompilerParams(use_tc_tiling_on_sc=False),
    )
    def gather_kernel(table_hbm, idx_hbm, out_hbm, idx_v, rows_v,
                      sem_g, sem_s, *part_refs):
        wid = lax.axis_index("s") * N_CORES + lax.axis_index("c")
        w0 = wid * span

        def load_remap_gather(slot, base, n):
            pltpu.sync_copy(idx_hbm.at[pl.ds(base, CH)], idx_v.at[slot])
            for i in range(CH // LANES):
                sl = pl.ds(i * LANES, LANES)
                idx_v[slot, sl] = idx_v[slot, sl] + 1
            pltpu.async_copy(table_hbm.at[idx_v.at[slot]],
                             rows_v.at[slot], sem_g.at[slot])

        def group(j, carry):
            for b in range(G):
                base = w0 + (j * G + b) * CH

                @pl.when(j > 0)
                def _():
                    pltpu.make_async_copy(
                        rows_v.at[b], out_hbm.at[pl.ds(base, CH)],
                        sem_s.at[b]).wait()

                load_remap_gather(b, base, CH)
            for b in range(G):
                base = w0 + (j * G + b) * CH
                pltpu.make_async_copy(
                    table_hbm.at[idx_v.at[b]], rows_v.at[b],
                    sem_g.at[b]).wait()
                pltpu.async_copy(rows_v.at[b],
                                 out_hbm.at[pl.ds(base, CH)],
                                 sem_s.at[b])
            return carry

        lax.fori_loop(0, n_main // G, group, 0, unroll=False)

        for b in range(G):
            pltpu.make_async_copy(
                rows_v.at[b], out_hbm.at[pl.ds(w0, CH)], sem_s.at[b]).wait()

        def full_chunk(slot, base):
            pltpu.sync_copy(idx_hbm.at[pl.ds(base, CH)], idx_v.at[slot])
            for i in range(CH // LANES):
                sl = pl.ds(i * LANES, LANES)
                idx_v[slot, sl] = idx_v[slot, sl] + 1
            pltpu.async_copy(table_hbm.at[idx_v.at[slot]],
                             rows_v.at[slot], sem_g.at[slot])

        def part_chunk(base, idx_p, rows_p, n):
            pltpu.sync_copy(idx_hbm.at[pl.ds(base, n)], idx_p)
            for i in range(n // LANES):
                sl = pl.ds(i * LANES, LANES)
                idx_p[sl] = idx_p[sl] + 1
            pltpu.async_copy(table_hbm.at[idx_p], rows_p,
                             sem_g.at[0]).wait()
            pltpu.sync_copy(rows_p, out_hbm.at[pl.ds(base, n)])

        part_refs = list(part_refs)
        hi_refs = [part_refs.pop(0), part_refs.pop(0)] if hi_part else None
        lo_refs = [part_refs.pop(0), part_refs.pop(0)] if lo_part else None

        @pl.when(wid < NW - 1)
        def _():
            e0 = w0 + n_main * CH
            for b in range(hi_full):
                full_chunk(b, e0 + b * CH)
            for b in range(hi_full):
                pltpu.make_async_copy(
                    table_hbm.at[idx_v.at[b]], rows_v.at[b],
                    sem_g.at[b]).wait()
                pltpu.sync_copy(rows_v.at[b],
                                out_hbm.at[pl.ds(e0 + b * CH, CH)])
            if hi_part:
                part_chunk(e0 + hi_full * CH, hi_refs[0], hi_refs[1],
                           hi_part)

        @pl.when(wid == NW - 1)
        def _():
            e0 = w0 + n_main * CH
            for b in range(lo_full):
                full_chunk(b, e0 + b * CH)
            for b in range(lo_full):
                pltpu.make_async_copy(
                    table_hbm.at[idx_v.at[b]], rows_v.at[b],
                    sem_g.at[b]).wait()
                pltpu.sync_copy(rows_v.at[b],
                                out_hbm.at[pl.ds(e0 + b * CH, CH)])
            if lo_part:
                part_chunk(e0 + lo_full * CH, lo_refs[0], lo_refs[1],
                           lo_part)

    return gather_kernel


def kernel(data, neigh, depth):
    n, k = neigh.shape
    c = data.shape[1]
    table = jnp.concatenate([jnp.zeros((1, c), data.dtype), data], axis=0)
    idx = neigh.reshape(-1).astype(jnp.int32)
    out_flat = _build_gather(n * k, c)(table, idx)
    return out_flat.reshape(n, k, c)


# speedup vs baseline: 7.7787x; 1.0461x over previous
import jax
import jax.numpy as jnp
from jax import lax
from jax.experimental import pallas as pl
from jax.experimental.pallas import tpu as pltpu
from jax.experimental.pallas import tpu_sc as plsc
import functools

N_CORES = 2
N_SUBCORES = 16
NW = N_CORES * N_SUBCORES
BN = 8
G = 8


def _build_gather(N: int, K: int, C: int):
    assert N % BN == 0
    ng = N // BN
    n_base, rem = divmod(ng, NW)
    n_groups = n_base // G
    n_main = n_groups * G

    mesh = plsc.VectorSubcoreMesh(
        core_axis_name="c", subcore_axis_name="s",
        num_cores=N_CORES, num_subcores=N_SUBCORES)

    @functools.partial(
        pl.kernel,
        out_type=jax.ShapeDtypeStruct((N, K, C), jnp.float32),
        mesh=mesh,
        scratch_types=[
            pltpu.VMEM((G, BN, K), jnp.int32),
            pltpu.VMEM((G, BN, K, C), jnp.float32),
            pltpu.SemaphoreType.DMA((G,)),
            pltpu.SemaphoreType.DMA((G,)),
        ],
        compiler_params=pltpu.CompilerParams(use_tc_tiling_on_sc=False),
    )
    def gather_kernel(table_hbm, idx_hbm, out_hbm, idx_v, rows_v,
                      sem_g, sem_s):
        wid = lax.axis_index("s") * N_CORES + lax.axis_index("c")
        n_my = n_base + (wid < rem).astype(jnp.int32)

        def gather_chunk(b, g):
            pltpu.sync_copy(idx_hbm.at[g], idx_v.at[b])
            for i in range(BN):
                pltpu.async_copy(table_hbm.at[idx_v.at[b, i]],
                                 rows_v.at[b, i], sem_g.at[b])

        def drain_gathers(b):
            for i in range(BN):
                pltpu.make_async_copy(table_hbm.at[idx_v.at[b, i]],
                                      rows_v.at[b, i], sem_g.at[b]).wait()

        def group(j, carry):
            for b in range(G):
                g = wid + (j * G + b) * NW

                @pl.when(j > 0)
                def _():
                    pltpu.make_async_copy(
                        rows_v.at[b], out_hbm.at[pl.ds(g * BN, BN)],
                        sem_s.at[b]).wait()

                gather_chunk(b, g)
            for b in range(G):
                g = wid + (j * G + b) * NW
                drain_gathers(b)
                pltpu.async_copy(rows_v.at[b],
                                 out_hbm.at[pl.ds(g * BN, BN)],
                                 sem_s.at[b])
            return carry

        lax.fori_loop(0, n_groups, group, 0, unroll=False)

        for b in range(G):
            pltpu.make_async_copy(
                rows_v.at[b], out_hbm.at[pl.ds(wid * BN, BN)],
                sem_s.at[b]).wait()

        def leftover(j, carry):
            g = wid + j * NW
            gather_chunk(0, g)
            drain_gathers(0)
            pltpu.sync_copy(rows_v.at[0], out_hbm.at[pl.ds(g * BN, BN)])
            return carry

        lax.fori_loop(n_main, n_my, leftover, 0, unroll=False)

    return gather_kernel


def kernel(data, neigh, depth):
    n, k = neigh.shape
    c = data.shape[1]
    table = jnp.concatenate([jnp.zeros((1, c), data.dtype), data], axis=0)
    idx = (neigh.astype(jnp.int32) + 1).reshape(n // BN, BN, k)
    return _build_gather(n, k, c)(table, idx)


# speedup vs baseline: 18.3931x; 2.3646x over previous
import jax
import jax.numpy as jnp
from jax import lax
from jax.experimental import pallas as pl
from jax.experimental.pallas import tpu as pltpu
from jax.experimental.pallas import tpu_sc as plsc
import functools

N_CORES = 2
N_SUBCORES = 16
NW = N_CORES * N_SUBCORES
BLK = 448
G = 4
L = 16


def _build_gather(N: int, k0: int, k1: int, C: int):
    K = k1 - k0
    nt_full, n_part = divmod(N, BLK)
    n_units = nt_full * K
    n_base, rem = divmod(n_units, NW)
    n_groups = n_base // G
    n_main = n_groups * G
    assert n_part % 8 == 0 and K <= NW

    mesh = plsc.VectorSubcoreMesh(
        core_axis_name="c", subcore_axis_name="s",
        num_cores=N_CORES, num_subcores=N_SUBCORES)

    @functools.partial(
        pl.kernel,
        out_type=jax.ShapeDtypeStruct((K, C, N), jnp.float32),
        mesh=mesh,
        scratch_types=[
            pltpu.VMEM((G, BLK), jnp.int32),
            pltpu.VMEM((G, BLK, C), jnp.float32),
            pltpu.VMEM((G, C, BLK + 1), jnp.float32),
            pltpu.SemaphoreType.DMA((G,)),
            pltpu.SemaphoreType.DMA((G,)),
        ],
        compiler_params=pltpu.CompilerParams(use_tc_tiling_on_sc=False, needs_layout_passes=False),
    )
    def gather_kernel(table_hbm, idxt_hbm, outt_hbm, idx_v, rows_v,
                      pan_v, sem_g, sem_s):
        wid = lax.axis_index("s") * N_CORES + lax.axis_index("c")
        n_my = n_base + (wid < rem).astype(jnp.int32)
        lane = lax.iota(jnp.int32, L)

        def unit_kn(u):
            return u % K, (u // K) * BLK

        def start_unit(b, u):
            k, n0 = unit_kn(u)
            pltpu.sync_copy(idxt_hbm.at[k0 + k, pl.ds(n0, BLK)], idx_v.at[b])

            @plsc.parallel_loop(0, BLK // L, unroll=4)
            def remap(t):
                sl = pl.ds(t * L, L)
                idx_v[b, sl] = idx_v[b, sl] + 1

            pltpu.async_copy(table_hbm.at[idx_v.at[b]], rows_v.at[b],
                             sem_g.at[b])

        c_idx = [lane + h * L for h in range(C // L)]

        def transpose_unit(b, nb):
            @plsc.parallel_loop(0, nb, unroll=4)
            def node(n):
                nvec = jnp.full((L,), n, jnp.int32)
                for h in range(C // L):
                    vals = rows_v[b, n, pl.ds(h * L, L)]
                    plsc.store_scatter(pan_v.at[b], [c_idx[h], nvec], vals)

        def panel_copy(u, b):
            k, n0 = unit_kn(u)
            return pltpu.make_async_copy(
                pan_v.at[b, :, pl.ds(0, BLK)],
                outt_hbm.at[k, :, pl.ds(n0, BLK)], sem_s.at[b])

        def group(j, carry):
            for b in range(G):
                u = wid + (j * G + b) * NW

                @pl.when(j > 0)
                def _():
                    panel_copy(u, b).wait()

                start_unit(b, u)
            for b in range(G):
                u = wid + (j * G + b) * NW
                pltpu.make_async_copy(table_hbm.at[idx_v.at[b]],
                                      rows_v.at[b], sem_g.at[b]).wait()
                transpose_unit(b, BLK)
                panel_copy(u, b).start()
            return carry

        lax.fori_loop(0, n_groups, group, 0, unroll=False)

        for b in range(G):
            pltpu.make_async_copy(
                pan_v.at[b, :, pl.ds(0, BLK)],
                outt_hbm.at[0, :, pl.ds(0, BLK)], sem_s.at[b]).wait()

        def leftover(j, carry):
            u = wid + j * NW
            start_unit(0, u)
            pltpu.make_async_copy(table_hbm.at[idx_v.at[0]],
                                  rows_v.at[0], sem_g.at[0]).wait()
            transpose_unit(0, BLK)
            k, n0 = unit_kn(u)
            pltpu.sync_copy(pan_v.at[0, :, pl.ds(0, BLK)],
                            outt_hbm.at[k, :, pl.ds(n0, BLK)])
            return carry

        lax.fori_loop(n_main, n_my, leftover, 0, unroll=False)

        if n_part:
            @pl.when(wid < K)
            def _():
                k = wid
                n0 = nt_full * BLK
                pltpu.sync_copy(idxt_hbm.at[k0 + k, pl.ds(n0, n_part)],
                                idx_v.at[0, pl.ds(0, n_part)])

                @plsc.parallel_loop(0, n_part // L, unroll=4)
                def remapp(t):
                    sl = pl.ds(t * L, L)
                    idx_v[0, sl] = idx_v[0, sl] + 1
                pltpu.async_copy(
                    table_hbm.at[idx_v.at[0, pl.ds(0, n_part)]],
                    rows_v.at[0, pl.ds(0, n_part)], sem_g.at[0]).wait()

                @plsc.parallel_loop(0, n_part, unroll=4)
                def nodep(n):
                    nvec = jnp.full((L,), n, jnp.int32)
                    for h in range(C // L):
                        vals = rows_v[0, n, pl.ds(h * L, L)]
                        plsc.store_scatter(pan_v.at[0], [c_idx[h], nvec], vals)
                pltpu.sync_copy(pan_v.at[0, :, pl.ds(0, n_part)],
                                outt_hbm.at[k, :, pl.ds(n0, n_part)])

    return gather_kernel


N_PARTS = 1


def kernel(data, neigh, depth):
    n, k = neigh.shape
    c = data.shape[1]
    table = jnp.concatenate([jnp.zeros((1, c), data.dtype), data], axis=0)
    idxt = neigh.T.astype(jnp.int32)
    bounds = [round(i * k / N_PARTS) for i in range(N_PARTS + 1)]
    parts = []
    for k0, k1 in zip(bounds[:-1], bounds[1:]):
        outt = _build_gather(n, k0, k1, c)(table, idxt)
        parts.append(jnp.transpose(outt, (2, 0, 1)))
    return parts[0] if len(parts) == 1 else jnp.concatenate(parts, axis=1)
